# baseline (speedup 1.0000x reference)
import functools

import jax
import jax.numpy as jnp
from jax import lax
from jax.experimental import pallas as pl
from jax.experimental.pallas import tpu as pltpu
from jax.experimental.pallas import tpu_sc as plsc

N = 10000
E = 320000
H = 128
NC = 2
NS = 16
NW = NC * NS
CH = 80
S = 2
NCH_SEG = (63, 62)
TPS_SEG = (63 * CH, 62 * CH)
SEG_E = (TPS_SEG[0] * NW, TPS_SEG[1] * NW)
SEG_OFF = (0, SEG_E[0])
EB_SEG = (4032, 3968)
NB = 400
NGRID = N // NB
PW = 16


def _sc_mesh():
    return plsc.VectorSubcoreMesh(core_axis_name="c", subcore_axis_name="s",
                                  num_cores=NC, num_subcores=NS)


_SC_PARAMS = pltpu.CompilerParams(needs_layout_passes=False)
_SC_PARAMS_NOTCTILE = pltpu.CompilerParams(needs_layout_passes=False,
                                           use_tc_tiling_on_sc=False)


def _emb_body(ids_ref, emb_ref, out_ref):
    ids = ids_ref[...]
    cols = lax.broadcasted_iota(jnp.int32, (NB, H), 1)
    onehot = (cols == ids).astype(jnp.float32)
    out_ref[...] = jnp.dot(onehot, emb_ref[...],
                           preferred_element_type=jnp.float32)


def _embed(ids2d, embp):
    return pl.pallas_call(
        _emb_body,
        grid=(NGRID,),
        in_specs=[pl.BlockSpec((NB, 1), lambda i: (i, 0)),
                  pl.BlockSpec((H, H), lambda i: (0, 0))],
        out_specs=pl.BlockSpec((NB, H), lambda i: (i, 0)),
        out_shape=jax.ShapeDtypeStruct((N, H), jnp.float32),
    )(ids2d, embp)


def _pack_bf16(h):
    hb = h.astype(jnp.bfloat16).reshape(N, H // 2, 2)
    return lax.bitcast_convert_type(hb, jnp.float32)


def _make_gather_body(seg):
    ncnk = NCH_SEG[seg]
    tps = TPS_SEG[seg]

    def body(h_hbm, xf_hbm, src_hbm, dst_hbm,
             hi_hbm, hj_hbm, g_hbm,
             xtab, sidx, didx,
             hibuf0, hjbuf0, gbuf0, hibuf1, hjbuf1, gbuf1,
             gsem0, gsem1, wsem0, wsem1):
        c = lax.axis_index("c")
        s = lax.axis_index("s")
        base = (c * NS + s) * tps
        inbase = SEG_OFF[seg] + base
        pltpu.sync_copy(xf_hbm, xtab)
        pltpu.sync_copy(src_hbm.at[pl.ds(inbase, tps)], sidx)
        pltpu.sync_copy(dst_hbm.at[pl.ds(inbase, tps)], didx)
        iota16 = lax.iota(jnp.int32, 16)
        hibuf = (hibuf0, hibuf1)
        hjbuf = (hjbuf0, hjbuf1)
        gbuf = (gbuf0, gbuf1)
        gsem = (gsem0, gsem1)
        wsem = (wsem0, wsem1)

        def ioff(j):
            return pl.multiple_of(j * CH, 8)

        def hoff(j):
            return pl.multiple_of(base + j * CH, 8)

        def issue_gather(j, b):
            pltpu.async_copy(h_hbm.at[sidx.at[pl.ds(ioff(j), CH)]],
                             hjbuf[b], gsem[b])
            pltpu.async_copy(h_hbm.at[didx.at[pl.ds(ioff(j), CH)]],
                             hibuf[b], gsem[b])

        def wait_gather(j, b):
            pltpu.make_async_copy(h_hbm.at[sidx.at[pl.ds(ioff(j), CH)]],
                                  hjbuf[b], gsem[b]).wait()
            pltpu.make_async_copy(h_hbm.at[didx.at[pl.ds(ioff(j), CH)]],
                                  hibuf[b], gsem[b]).wait()

        def issue_wb(j, b):
            pltpu.async_copy(hibuf[b], hi_hbm.at[pl.ds(hoff(j), CH)], wsem[b])
            pltpu.async_copy(hjbuf[b], hj_hbm.at[pl.ds(hoff(j), CH)], wsem[b])
            pltpu.async_copy(gbuf[b], g_hbm.at[pl.ds(hoff(j), CH)], wsem[b])

        def wait_wb(j, b):
            pltpu.make_async_copy(hibuf[b], hi_hbm.at[pl.ds(hoff(j), CH)],
                                  wsem[b]).wait()
            pltpu.make_async_copy(hjbuf[b], hj_hbm.at[pl.ds(hoff(j), CH)],
                                  wsem[b]).wait()
            pltpu.make_async_copy(gbuf[b], g_hbm.at[pl.ds(hoff(j), CH)],
                                  wsem[b]).wait()

        def sync_wb(j, b):
            pltpu.sync_copy(hibuf[b], hi_hbm.at[pl.ds(hoff(j), CH)])
            pltpu.sync_copy(hjbuf[b], hj_hbm.at[pl.ds(hoff(j), CH)])
            pltpu.sync_copy(gbuf[b], g_hbm.at[pl.ds(hoff(j), CH)])

        def compute_g(j, b):
            for q in range(CH // 16):
                o = pl.multiple_of(j * CH, 8) + q * 16
                s16 = sidx[pl.ds(o, 16)]
                d16 = didx[pl.ds(o, 16)]
                comps = []
                for cc in range(3):
                    xs = plsc.load_gather(xtab, [s16 * 4 + cc])
                    xd = plsc.load_gather(xtab, [d16 * 4 + cc])
                    comps.append(xs - xd)
                r2 = (comps[0] * comps[0] + comps[1] * comps[1]
                      + comps[2] * comps[2])
                r16 = q * 16 + iota16
                for cc in range(3):
                    plsc.store_scatter(gbuf[b], [r16, iota16 * 0 + cc],
                                       comps[cc])
                plsc.store_scatter(gbuf[b], [r16, iota16 * 0 + 3], r2)

        issue_gather(0, 0)
        npairs = (ncnk - 1) // 2
        rem = ncnk - 2 * npairs

        def dbl(jj, carry):
            j0 = jj * 2
            wait_gather(j0, 0)
            compute_g(j0, 0)

            @pl.when(jj >= 1)
            def _():
                wait_wb(j0 - 1, 1)

            issue_gather(j0 + 1, 1)
            issue_wb(j0, 0)

            j1 = j0 + 1
            wait_gather(j1, 1)
            compute_g(j1, 1)
            wait_wb(j1 - 1, 0)
            issue_gather(j1 + 1, 0)
            issue_wb(j1, 1)
            return carry

        lax.fori_loop(0, npairs, dbl, 0)
        c0 = 2 * npairs
        if rem == 1:
            wait_gather(c0, 0)
            compute_g(c0, 0)
            wait_wb(c0 - 1, 1)
            sync_wb(c0, 0)
        else:
            c1 = c0 + 1
            wait_gather(c0, 0)
            compute_g(c0, 0)
            wait_wb(c0 - 1, 1)
            issue_gather(c1, 1)
            issue_wb(c0, 0)
            wait_gather(c1, 1)
            compute_g(c1, 1)
            wait_wb(c0, 0)
            sync_wb(c1, 1)

    return body


def _sc_gather(h, xflat, src, dst, seg):
    tps = TPS_SEG[seg]
    f = pl.kernel(
        _make_gather_body(seg),
        out_type=(jax.ShapeDtypeStruct((SEG_E[seg], H // 2), jnp.float32),
                  jax.ShapeDtypeStruct((SEG_E[seg], H // 2), jnp.float32),
                  jax.ShapeDtypeStruct((SEG_E[seg], 8), jnp.float32)),
        mesh=_sc_mesh(),
        compiler_params=_SC_PARAMS_NOTCTILE,
        scratch_types=[pltpu.VMEM((4 * N,), jnp.float32),
                       pltpu.VMEM((tps,), jnp.int32),
                       pltpu.VMEM((tps,), jnp.int32),
                       pltpu.VMEM((CH, H // 2), jnp.float32),
                       pltpu.VMEM((CH, H // 2), jnp.float32),
                       pltpu.VMEM((CH, 8), jnp.float32),
                       pltpu.VMEM((CH, H // 2), jnp.float32),
                       pltpu.VMEM((CH, H // 2), jnp.float32),
                       pltpu.VMEM((CH, 8), jnp.float32),
                       pltpu.SemaphoreType.DMA,
                       pltpu.SemaphoreType.DMA,
                       pltpu.SemaphoreType.DMA,
                       pltpu.SemaphoreType.DMA],
    )
    return f(h, xflat, src, dst)


def _unpack_bf16_pair(w):
    u = pltpu.bitcast(w, jnp.uint32)
    lo = pltpu.bitcast(u << 16, jnp.float32)
    hi = pltpu.bitcast(u & jnp.uint32(0xFFFF0000), jnp.float32)
    return lo, hi


def _make_edge_body(EB):
    def body(hi_ref, hj_ref, g_ref, Ae_ref, Ao_ref, Be_ref, Bo_ref,
             c_ref, b1_ref,
             W2_ref, b2_ref, xw1_ref, xb1_ref, xw2_ref, xb2_ref,
             mij_ref, p_ref):
        g = g_ref[...]
        r = jnp.sqrt(g[:, 3:4])
        hie, hio = _unpack_bf16_pair(hi_ref[...])
        hje, hjo = _unpack_bf16_pair(hj_ref[...])
        t = (jnp.dot(hie, Ae_ref[...], preferred_element_type=jnp.float32)
             + jnp.dot(hio, Ao_ref[...], preferred_element_type=jnp.float32)
             + jnp.dot(hje, Be_ref[...], preferred_element_type=jnp.float32)
             + jnp.dot(hjo, Bo_ref[...], preferred_element_type=jnp.float32)
             + r * c_ref[...] + b1_ref[...])
        m = t * jax.nn.sigmoid(t)
        t2 = (jnp.dot(m, W2_ref[...], preferred_element_type=jnp.float32)
              + b2_ref[...])
        m2 = t2 * jax.nn.sigmoid(t2)
        mij_ref[...] = m2
        t3 = (jnp.dot(m2, xw1_ref[...], preferred_element_type=jnp.float32)
              + xb1_ref[...])
        w = t3 * jax.nn.sigmoid(t3)
        Wij = (jnp.dot(w, xw2_ref[...], preferred_element_type=jnp.float32)
               + xb2_ref[...])
        p = Wij * g[:, 0:3]
        p_ref[...] = jnp.concatenate(
            [p, jnp.zeros((EB, PW - 3), jnp.float32)], axis=1)

    return body


def _edge_mlp(hi, hj, g, Ae, Ao, Be, Bo, crow, b1, W2, b2,
              xw1, xb1, xw2, xb2, seg):
    EB = EB_SEG[seg]
    ngrid = SEG_E[seg] // EB
    out_rows = SEG_E[seg]
    H2 = H // 2
    full = lambda shape: pl.BlockSpec(shape, lambda i: tuple(0 for _ in shape))
    return pl.pallas_call(
        _make_edge_body(EB),
        grid=(ngrid,),
        in_specs=[pl.BlockSpec((EB, H2), lambda i: (i, 0)),
                  pl.BlockSpec((EB, H2), lambda i: (i, 0)),
                  pl.BlockSpec((EB, 8), lambda i: (i, 0)),
                  full((H2, H)), full((H2, H)), full((H2, H)), full((H2, H)),
                  full((1, H)), full((1, H)),
                  full((H, H)), full((1, H)),
                  full((H, H)), full((1, H)), full((H, 1)), full((1, 1))],
        out_specs=[pl.BlockSpec((EB, H), lambda i: (i, 0)),
                   pl.BlockSpec((EB, PW), lambda i: (i, 0))],
        out_shape=[jax.ShapeDtypeStruct((out_rows, H), jnp.float32),
                   jax.ShapeDtypeStruct((out_rows, PW), jnp.float32)],
    )(hi, hj, g, Ae, Ao, Be, Bo, crow, b1, W2, b2, xw1, xb1, xw2, xb2)


def _make_scatter_m_body(seg):
    ncnk = NCH_SEG[seg]
    tps = TPS_SEG[seg]

    def body(mij_hbm, dst3_hbm, zh_hbm, mpart_hbm,
             didx2, mbuf0, mbuf1, lsem0, lsem1, macc):
        c = lax.axis_index("c")
        s = lax.axis_index("s")
        wid = c * NS + s
        base = wid * tps
        rows0 = s * 624

        @pl.when(s < NS - 1)
        def _():
            pltpu.sync_copy(zh_hbm.at[pl.ds(rows0, 624)],
                            macc.at[pl.ds(rows0, 624)])

        @pl.when(s == NS - 1)
        def _():
            pltpu.sync_copy(zh_hbm.at[pl.ds(rows0, 640)],
                            macc.at[pl.ds(rows0, 640)])

        pltpu.sync_copy(dst3_hbm.at[wid], didx2)
        plsc.subcore_barrier()
        mbuf = (mbuf0, mbuf1)
        lsem = (lsem0, lsem1)

        def hoff(j):
            return pl.multiple_of(base + j * CH, 8)

        def issue_load(j, b):
            pltpu.async_copy(mij_hbm.at[pl.ds(hoff(j), CH)], mbuf[b], lsem[b])

        def wait_load(j, b):
            pltpu.make_async_copy(mij_hbm.at[pl.ds(hoff(j), CH)],
                                  mbuf[b], lsem[b]).wait()

        issue_load(0, 0)
        npairs = (ncnk - 1) // 2
        rem = ncnk - 2 * npairs

        def dbl(jj, carry):
            j0 = jj * 2
            issue_load(j0 + 1, 1)
            wait_load(j0, 0)
            pltpu.sync_copy(mbuf[0], macc.at[didx2.at[j0]], add=True)
            j1 = j0 + 1
            issue_load(j1 + 1, 0)
            wait_load(j1, 1)
            pltpu.sync_copy(mbuf[1], macc.at[didx2.at[j1]], add=True)
            return carry

        lax.fori_loop(0, npairs, dbl, 0)
        c0 = 2 * npairs
        if rem == 1:
            wait_load(c0, 0)
            pltpu.sync_copy(mbuf[0], macc.at[didx2.at[c0]], add=True)
        else:
            c1 = c0 + 1
            issue_load(c1, 1)
            wait_load(c0, 0)
            pltpu.sync_copy(mbuf[0], macc.at[didx2.at[c0]], add=True)
            wait_load(c1, 1)
            pltpu.sync_copy(mbuf[1], macc.at[didx2.at[c1]], add=True)
        plsc.subcore_barrier()

        @pl.when(s < NS - 1)
        def _():
            pltpu.sync_copy(macc.at[pl.ds(rows0, 624)],
                            mpart_hbm.at[c].at[pl.ds(rows0, 624)])

        @pl.when(s == NS - 1)
        def _():
            pltpu.sync_copy(macc.at[pl.ds(rows0, 640)],
                            mpart_hbm.at[c].at[pl.ds(rows0, 640)])

    return body


def _sc_scatter_m(mij, dst3, zh, seg):
    ncnk = NCH_SEG[seg]
    f = pl.kernel(
        _make_scatter_m_body(seg),
        out_type=jax.ShapeDtypeStruct((NC, N, H), jnp.float32),
        mesh=_sc_mesh(),
        compiler_params=_SC_PARAMS,
        scratch_types=[pltpu.VMEM((ncnk, CH), jnp.int32),
                       pltpu.VMEM((CH, H), jnp.float32),
                       pltpu.VMEM((CH, H), jnp.float32),
                       pltpu.SemaphoreType.DMA,
                       pltpu.SemaphoreType.DMA,
                       pltpu.VMEM_SHARED((N, H), jnp.float32)],
    )
    return f(mij, dst3, zh)


def _make_scatter_x_body(seg):
    ncnk = NCH_SEG[seg]
    tps = TPS_SEG[seg]

    def body(pf_hbm, dst_hbm, xpart_hbm,
             didx, pbuf0, pbuf1, psem0, psem1, xacc):
        c = lax.axis_index("c")
        s = lax.axis_index("s")
        wid = c * NS + s
        base = wid * tps
        iota16 = lax.iota(jnp.int32, 16)
        pltpu.sync_copy(dst_hbm.at[pl.ds(SEG_OFF[seg] + base, tps)], didx)

        def zloop(i, carry):
            plsc.store_scatter(xacc, [i * 16 + iota16],
                               jnp.zeros((16,), jnp.float32))
            return carry

        lax.fori_loop(0, (N * 4) // 16, zloop, 0)
        pbuf = (pbuf0, pbuf1)
        psem = (psem0, psem1)

        def hoff(j):
            return pl.multiple_of(base + j * CH, 8)

        def issue_load(j, b):
            pltpu.async_copy(pf_hbm.at[pl.ds(hoff(j), CH)], pbuf[b], psem[b])

        def wait_load(j, b):
            pltpu.make_async_copy(pf_hbm.at[pl.ds(hoff(j), CH)],
                                  pbuf[b], psem[b]).wait()

        def compute(j, b):
            for q in range(CH // 16):
                o = pl.multiple_of(j * CH, 8) + q * 16
                d16 = didx[pl.ds(o, 16)]
                r16 = q * 16 + iota16
                for cc in range(3):
                    val = plsc.load_gather(pbuf[b], [r16, iota16 * 0 + cc])
                    plsc.addupdate_scatter(xacc, [d16 * 4 + cc], val)

        issue_load(0, 0)
        npairs = (ncnk - 1) // 2
        rem = ncnk - 2 * npairs

        def dbl(jj, carry):
            j0 = jj * 2
            issue_load(j0 + 1, 1)
            wait_load(j0, 0)
            compute(j0, 0)
            j1 = j0 + 1
            issue_load(j1 + 1, 0)
            wait_load(j1, 1)
            compute(j1, 1)
            return carry

        lax.fori_loop(0, npairs, dbl, 0)
        c0 = 2 * npairs
        if rem == 1:
            wait_load(c0, 0)
            compute(c0, 0)
        else:
            c1 = c0 + 1
            issue_load(c1, 1)
            wait_load(c0, 0)
            compute(c0, 0)
            wait_load(c1, 1)
            compute(c1, 1)
        pltpu.sync_copy(xacc, xpart_hbm.at[wid])

    return body


def _sc_scatter_x(parr, dst, seg):
    tps = TPS_SEG[seg]
    f = pl.kernel(
        _make_scatter_x_body(seg),
        out_type=jax.ShapeDtypeStruct((NW, N * 4), jnp.float32),
        mesh=_sc_mesh(),
        compiler_params=_SC_PARAMS,
        scratch_types=[pltpu.VMEM((tps,), jnp.int32),
                       pltpu.VMEM((CH, PW), jnp.float32),
                       pltpu.VMEM((CH, PW), jnp.float32),
                       pltpu.SemaphoreType.DMA,
                       pltpu.SemaphoreType.DMA,
                       pltpu.VMEM((N * 4,), jnp.float32)],
    )
    return f(parr, dst)


def _xsum_body(x_ref, xp0_ref, xp1_ref, out_ref):
    out_ref[...] = (x_ref[...] + jnp.sum(xp0_ref[...], axis=0)
                    + jnp.sum(xp1_ref[...], axis=0))


def _xsum(x4, xpart0, xpart1):
    XL = 1600
    x3 = x4.reshape(NGRID, 1, XL)
    xp0 = xpart0.reshape(NW, NGRID, 1, XL)
    xp1 = xpart1.reshape(NW, NGRID, 1, XL)
    out = pl.pallas_call(
        _xsum_body,
        grid=(NGRID,),
        in_specs=[pl.BlockSpec((1, 1, XL), lambda i: (i, 0, 0)),
                  pl.BlockSpec((NW, 1, 1, XL), lambda i: (0, i, 0, 0)),
                  pl.BlockSpec((NW, 1, 1, XL), lambda i: (0, i, 0, 0))],
        out_specs=pl.BlockSpec((1, 1, XL), lambda i: (i, 0, 0)),
        out_shape=jax.ShapeDtypeStruct((NGRID, 1, XL), jnp.float32),
    )(x3, xp0, xp1)
    return out.reshape(N, 4)


def _node_body(h_ref, mp0_ref, mp1_ref, U_ref, V_ref, b1_ref,
               W2_ref, b2_ref, hout_ref):
    m_i = mp0_ref[0] + mp0_ref[1] + mp1_ref[0] + mp1_ref[1]
    t = (jnp.dot(h_ref[...], U_ref[...], preferred_element_type=jnp.float32)
         + jnp.dot(m_i, V_ref[...], preferred_element_type=jnp.float32)
         + b1_ref[...])
    hh = t * jax.nn.sigmoid(t)
    hout_ref[...] = (h_ref[...]
                     + jnp.dot(hh, W2_ref[...], preferred_element_type=jnp.float32)
                     + b2_ref[...])


def _node_update(h, mpart0, mpart1, U, V, hb1, hW2, hb2):
    full = lambda shape: pl.BlockSpec(shape, lambda i: tuple(0 for _ in shape))
    return pl.pallas_call(
        _node_body,
        grid=(NGRID,),
        in_specs=[pl.BlockSpec((NB, H), lambda i: (i, 0)),
                  pl.BlockSpec((NC, NB, H), lambda i: (0, i, 0)),
                  pl.BlockSpec((NC, NB, H), lambda i: (0, i, 0)),
                  full((H, H)), full((H, H)), full((1, H)),
                  full((H, H)), full((1, H))],
        out_specs=pl.BlockSpec((NB, H), lambda i: (i, 0)),
        out_shape=jax.ShapeDtypeStruct((N, H), jnp.float32),
    )(h, mpart0, mpart1, U, V, hb1, hW2, hb2)


def kernel(atomic_numbers, pos, edge_index, edge_attr, emb,
           e_w1, e_b1, e_w2, e_b2,
           h_w1, h_b1, h_w2, h_b2,
           x_w1, x_b1, x_w2, x_b2):
    del edge_attr
    ids2d = atomic_numbers.astype(jnp.int32).reshape(N, 1)
    embp = jnp.zeros((H, H), jnp.float32).at[:emb.shape[0]].set(emb)
    src = edge_index[0].astype(jnp.int32)
    dst = edge_index[1].astype(jnp.int32)
    dst3 = [dst[SEG_OFF[s]:SEG_OFF[s] + SEG_E[s]].reshape(NW, NCH_SEG[s], CH)
            for s in range(S)]
    zh = jnp.zeros((N, H), jnp.float32)

    h = _embed(ids2d, embp)
    x4 = jnp.pad(pos, ((0, 0), (0, 1)))

    for l in range(e_w1.shape[0]):
        A = e_w1[l, :H]
        B = e_w1[l, H:2 * H]
        Ae, Ao = A[0::2], A[1::2]
        Be, Bo = B[0::2], B[1::2]
        crow = e_w1[l, 2 * H:2 * H + 1]
        b1 = e_b1[l].reshape(1, H)
        W2 = e_w2[l]
        b2 = e_b2[l].reshape(1, H)
        xw1 = x_w1[l]
        xb1 = x_b1[l].reshape(1, H)
        xw2 = x_w2[l]
        xb2 = x_b2[l].reshape(1, 1)
        U = h_w1[l, :H]
        V = h_w1[l, H:]
        hb1 = h_b1[l].reshape(1, H)
        hW2 = h_w2[l]
        hb2 = h_b2[l].reshape(1, H)

        xflat = x4.reshape(-1)
        hbp = _pack_bf16(h)
        hi0, hj0, g0 = _sc_gather(hbp, xflat, src, dst, 0)
        hi1, hj1, g1 = _sc_gather(hbp, xflat, src, dst, 1)
        mij0, parr0 = _edge_mlp(hi0, hj0, g0, Ae, Ao, Be, Bo, crow, b1,
                                W2, b2, xw1, xb1, xw2, xb2, 0)
        mij1, parr1 = _edge_mlp(hi1, hj1, g1, Ae, Ao, Be, Bo, crow, b1,
                                W2, b2, xw1, xb1, xw2, xb2, 1)
        mp0 = _sc_scatter_m(mij0, dst3[0], zh, 0)
        xp0 = _sc_scatter_x(parr0, dst, 0)
        mp1 = _sc_scatter_m(mij1, dst3[1], zh, 1)
        xp1 = _sc_scatter_x(parr1, dst, 1)
        x4 = _xsum(x4, xp0, xp1)
        h = _node_update(h, mp0, mp1, U, V, hb1, hW2, hb2)

    return (h, x4[:, :3])


# speedup vs baseline: 1.2765x; 1.2765x over previous
import functools

import jax
import jax.numpy as jnp
from jax import lax
from jax.experimental import pallas as pl
from jax.experimental.pallas import tpu as pltpu
from jax.experimental.pallas import tpu_sc as plsc

N = 10000
E = 320000
H = 128
NC = 2
NS = 16
NW = NC * NS
CH = 80
S = 2
NCH_SEG = (63, 62)
TPS_SEG = (63 * CH, 62 * CH)
SEG_E = (TPS_SEG[0] * NW, TPS_SEG[1] * NW)
SEG_OFF = (0, SEG_E[0])
EB_SEG = (4032, 3968)
NB = 400
NGRID = N // NB
PW = 16


def _sc_mesh():
    return plsc.VectorSubcoreMesh(core_axis_name="c", subcore_axis_name="s",
                                  num_cores=NC, num_subcores=NS)


_SC_PARAMS = pltpu.CompilerParams(needs_layout_passes=False)


def _emb_body(ids_ref, emb_ref, out_ref):
    ids = ids_ref[...]
    cols = lax.broadcasted_iota(jnp.int32, (NB, H), 1)
    onehot = (cols == ids).astype(jnp.float32)
    out_ref[...] = jnp.dot(onehot, emb_ref[...],
                           preferred_element_type=jnp.float32)


def _embed(ids2d, embp):
    return pl.pallas_call(
        _emb_body,
        grid=(NGRID,),
        in_specs=[pl.BlockSpec((NB, 1), lambda i: (i, 0)),
                  pl.BlockSpec((H, H), lambda i: (0, 0))],
        out_specs=pl.BlockSpec((NB, H), lambda i: (i, 0)),
        out_shape=jax.ShapeDtypeStruct((N, H), jnp.float32),
    )(ids2d, embp)


def _make_gather_body(seg):
    ncnk = NCH_SEG[seg]
    tps = TPS_SEG[seg]

    def body(h_hbm, xf_hbm, src_hbm, dst_hbm,
             hi_hbm, hj_hbm, g_hbm,
             xtab, sidx, didx,
             hibuf0, hjbuf0, gbuf0, hibuf1, hjbuf1, gbuf1,
             gsem0, gsem1, wsem0, wsem1):
        c = lax.axis_index("c")
        s = lax.axis_index("s")
        base = (c * NS + s) * tps
        inbase = SEG_OFF[seg] + base
        pltpu.sync_copy(xf_hbm, xtab)
        pltpu.sync_copy(src_hbm.at[pl.ds(inbase, tps)], sidx)
        pltpu.sync_copy(dst_hbm.at[pl.ds(inbase, tps)], didx)
        iota16 = lax.iota(jnp.int32, 16)
        hibuf = (hibuf0, hibuf1)
        hjbuf = (hjbuf0, hjbuf1)
        gbuf = (gbuf0, gbuf1)
        gsem = (gsem0, gsem1)
        wsem = (wsem0, wsem1)

        def ioff(j):
            return pl.multiple_of(j * CH, 8)

        def hoff(j):
            return pl.multiple_of(base + j * CH, 8)

        def issue_gather(j, b):
            pltpu.async_copy(h_hbm.at[sidx.at[pl.ds(ioff(j), CH)]],
                             hjbuf[b], gsem[b])
            pltpu.async_copy(h_hbm.at[didx.at[pl.ds(ioff(j), CH)]],
                             hibuf[b], gsem[b])

        def wait_gather(j, b):
            pltpu.make_async_copy(h_hbm.at[sidx.at[pl.ds(ioff(j), CH)]],
                                  hjbuf[b], gsem[b]).wait()
            pltpu.make_async_copy(h_hbm.at[didx.at[pl.ds(ioff(j), CH)]],
                                  hibuf[b], gsem[b]).wait()

        def issue_wb(j, b):
            pltpu.async_copy(hibuf[b], hi_hbm.at[pl.ds(hoff(j), CH)], wsem[b])
            pltpu.async_copy(hjbuf[b], hj_hbm.at[pl.ds(hoff(j), CH)], wsem[b])
            pltpu.async_copy(gbuf[b], g_hbm.at[pl.ds(hoff(j), CH)], wsem[b])

        def wait_wb(j, b):
            pltpu.make_async_copy(hibuf[b], hi_hbm.at[pl.ds(hoff(j), CH)],
                                  wsem[b]).wait()
            pltpu.make_async_copy(hjbuf[b], hj_hbm.at[pl.ds(hoff(j), CH)],
                                  wsem[b]).wait()
            pltpu.make_async_copy(gbuf[b], g_hbm.at[pl.ds(hoff(j), CH)],
                                  wsem[b]).wait()

        def sync_wb(j, b):
            pltpu.sync_copy(hibuf[b], hi_hbm.at[pl.ds(hoff(j), CH)])
            pltpu.sync_copy(hjbuf[b], hj_hbm.at[pl.ds(hoff(j), CH)])
            pltpu.sync_copy(gbuf[b], g_hbm.at[pl.ds(hoff(j), CH)])

        def compute_g(j, b):
            for q in range(CH // 16):
                o = pl.multiple_of(j * CH, 8) + q * 16
                s16 = sidx[pl.ds(o, 16)]
                d16 = didx[pl.ds(o, 16)]
                comps = []
                for cc in range(3):
                    xs = plsc.load_gather(xtab, [s16 * 4 + cc])
                    xd = plsc.load_gather(xtab, [d16 * 4 + cc])
                    comps.append(xs - xd)
                r2 = (comps[0] * comps[0] + comps[1] * comps[1]
                      + comps[2] * comps[2])
                r16 = q * 16 + iota16
                for cc in range(3):
                    plsc.store_scatter(gbuf[b], [r16, iota16 * 0 + cc],
                                       comps[cc])
                plsc.store_scatter(gbuf[b], [r16, iota16 * 0 + 3], r2)

        issue_gather(0, 0)
        npairs = (ncnk - 1) // 2
        rem = ncnk - 2 * npairs

        def dbl(jj, carry):
            j0 = jj * 2
            wait_gather(j0, 0)
            compute_g(j0, 0)

            @pl.when(jj >= 1)
            def _():
                wait_wb(j0 - 1, 1)

            issue_gather(j0 + 1, 1)
            issue_wb(j0, 0)

            j1 = j0 + 1
            wait_gather(j1, 1)
            compute_g(j1, 1)
            wait_wb(j1 - 1, 0)
            issue_gather(j1 + 1, 0)
            issue_wb(j1, 1)
            return carry

        lax.fori_loop(0, npairs, dbl, 0)
        c0 = 2 * npairs
        if rem == 1:
            wait_gather(c0, 0)
            compute_g(c0, 0)
            wait_wb(c0 - 1, 1)
            sync_wb(c0, 0)
        else:
            c1 = c0 + 1
            wait_gather(c0, 0)
            compute_g(c0, 0)
            wait_wb(c0 - 1, 1)
            issue_gather(c1, 1)
            issue_wb(c0, 0)
            wait_gather(c1, 1)
            compute_g(c1, 1)
            wait_wb(c0, 0)
            sync_wb(c1, 1)

    return body


def _sc_gather(h, xflat, src, dst, seg):
    tps = TPS_SEG[seg]
    f = pl.kernel(
        _make_gather_body(seg),
        out_type=(jax.ShapeDtypeStruct((SEG_E[seg], H), jnp.float32),
                  jax.ShapeDtypeStruct((SEG_E[seg], H), jnp.float32),
                  jax.ShapeDtypeStruct((SEG_E[seg], 8), jnp.float32)),
        mesh=_sc_mesh(),
        compiler_params=_SC_PARAMS,
        scratch_types=[pltpu.VMEM((4 * N,), jnp.float32),
                       pltpu.VMEM((tps,), jnp.int32),
                       pltpu.VMEM((tps,), jnp.int32),
                       pltpu.VMEM((CH, H), jnp.float32),
                       pltpu.VMEM((CH, H), jnp.float32),
                       pltpu.VMEM((CH, 8), jnp.float32),
                       pltpu.VMEM((CH, H), jnp.float32),
                       pltpu.VMEM((CH, H), jnp.float32),
                       pltpu.VMEM((CH, 8), jnp.float32),
                       pltpu.SemaphoreType.DMA,
                       pltpu.SemaphoreType.DMA,
                       pltpu.SemaphoreType.DMA,
                       pltpu.SemaphoreType.DMA],
    )
    return f(h, xflat, src, dst)


def _make_edge_body(EB):
    def body(hi_ref, hj_ref, g_ref, A_ref, B_ref,
             c_ref, b1_ref,
             W2_ref, b2_ref, xw1_ref, xb1_ref, xw2_ref, xb2_ref,
             mij_ref, p_ref):
        g = g_ref[...]
        r = jnp.sqrt(g[:, 3:4])
        t = (jnp.dot(hi_ref[...], A_ref[...],
                     preferred_element_type=jnp.float32)
             + jnp.dot(hj_ref[...], B_ref[...],
                       preferred_element_type=jnp.float32)
             + r * c_ref[...] + b1_ref[...])
        m = t * jax.nn.sigmoid(t)
        t2 = (jnp.dot(m, W2_ref[...], preferred_element_type=jnp.float32)
              + b2_ref[...])
        m2 = t2 * jax.nn.sigmoid(t2)
        mij_ref[...] = m2
        t3 = (jnp.dot(m2, xw1_ref[...], preferred_element_type=jnp.float32)
              + xb1_ref[...])
        w = t3 * jax.nn.sigmoid(t3)
        Wij = (jnp.dot(w, xw2_ref[...], preferred_element_type=jnp.float32)
               + xb2_ref[...])
        p = Wij * g[:, 0:3]
        p_ref[...] = jnp.concatenate(
            [p, jnp.zeros((EB, PW - 3), jnp.float32)], axis=1)

    return body


def _edge_mlp(hi, hj, g, A, B, crow, b1, W2, b2,
              xw1, xb1, xw2, xb2, seg):
    EB = EB_SEG[seg]
    ngrid = SEG_E[seg] // EB
    out_rows = SEG_E[seg]
    full = lambda shape: pl.BlockSpec(shape, lambda i: tuple(0 for _ in shape))
    return pl.pallas_call(
        _make_edge_body(EB),
        grid=(ngrid,),
        in_specs=[pl.BlockSpec((EB, H), lambda i: (i, 0)),
                  pl.BlockSpec((EB, H), lambda i: (i, 0)),
                  pl.BlockSpec((EB, 8), lambda i: (i, 0)),
                  full((H, H)), full((H, H)),
                  full((1, H)), full((1, H)),
                  full((H, H)), full((1, H)),
                  full((H, H)), full((1, H)), full((H, 1)), full((1, 1))],
        out_specs=[pl.BlockSpec((EB, H), lambda i: (i, 0)),
                   pl.BlockSpec((EB, PW), lambda i: (i, 0))],
        out_shape=[jax.ShapeDtypeStruct((out_rows, H), jnp.float32),
                   jax.ShapeDtypeStruct((out_rows, PW), jnp.float32)],
    )(hi, hj, g, A, B, crow, b1, W2, b2, xw1, xb1, xw2, xb2)


def _make_scatter_m_body(seg):
    ncnk = NCH_SEG[seg]
    tps = TPS_SEG[seg]

    def body(mij_hbm, dst3_hbm, zh_hbm, mpart_hbm,
             didx2, mbuf0, mbuf1, lsem0, lsem1, macc):
        c = lax.axis_index("c")
        s = lax.axis_index("s")
        wid = c * NS + s
        base = wid * tps
        rows0 = s * 624

        @pl.when(s < NS - 1)
        def _():
            pltpu.sync_copy(zh_hbm.at[pl.ds(rows0, 624)],
                            macc.at[pl.ds(rows0, 624)])

        @pl.when(s == NS - 1)
        def _():
            pltpu.sync_copy(zh_hbm.at[pl.ds(rows0, 640)],
                            macc.at[pl.ds(rows0, 640)])

        pltpu.sync_copy(dst3_hbm.at[wid], didx2)
        plsc.subcore_barrier()
        mbuf = (mbuf0, mbuf1)
        lsem = (lsem0, lsem1)

        def hoff(j):
            return pl.multiple_of(base + j * CH, 8)

        def issue_load(j, b):
            pltpu.async_copy(mij_hbm.at[pl.ds(hoff(j), CH)], mbuf[b], lsem[b])

        def wait_load(j, b):
            pltpu.make_async_copy(mij_hbm.at[pl.ds(hoff(j), CH)],
                                  mbuf[b], lsem[b]).wait()

        issue_load(0, 0)
        npairs = (ncnk - 1) // 2
        rem = ncnk - 2 * npairs

        def dbl(jj, carry):
            j0 = jj * 2
            issue_load(j0 + 1, 1)
            wait_load(j0, 0)
            pltpu.sync_copy(mbuf[0], macc.at[didx2.at[j0]], add=True)
            j1 = j0 + 1
            issue_load(j1 + 1, 0)
            wait_load(j1, 1)
            pltpu.sync_copy(mbuf[1], macc.at[didx2.at[j1]], add=True)
            return carry

        lax.fori_loop(0, npairs, dbl, 0)
        c0 = 2 * npairs
        if rem == 1:
            wait_load(c0, 0)
            pltpu.sync_copy(mbuf[0], macc.at[didx2.at[c0]], add=True)
        else:
            c1 = c0 + 1
            issue_load(c1, 1)
            wait_load(c0, 0)
            pltpu.sync_copy(mbuf[0], macc.at[didx2.at[c0]], add=True)
            wait_load(c1, 1)
            pltpu.sync_copy(mbuf[1], macc.at[didx2.at[c1]], add=True)
        plsc.subcore_barrier()

        @pl.when(s < NS - 1)
        def _():
            pltpu.sync_copy(macc.at[pl.ds(rows0, 624)],
                            mpart_hbm.at[c].at[pl.ds(rows0, 624)])

        @pl.when(s == NS - 1)
        def _():
            pltpu.sync_copy(macc.at[pl.ds(rows0, 640)],
                            mpart_hbm.at[c].at[pl.ds(rows0, 640)])

    return body


def _sc_scatter_m(mij, dst3, zh, seg):
    ncnk = NCH_SEG[seg]
    f = pl.kernel(
        _make_scatter_m_body(seg),
        out_type=jax.ShapeDtypeStruct((NC, N, H), jnp.float32),
        mesh=_sc_mesh(),
        compiler_params=_SC_PARAMS,
        scratch_types=[pltpu.VMEM((ncnk, CH), jnp.int32),
                       pltpu.VMEM((CH, H), jnp.float32),
                       pltpu.VMEM((CH, H), jnp.float32),
                       pltpu.SemaphoreType.DMA,
                       pltpu.SemaphoreType.DMA,
                       pltpu.VMEM_SHARED((N, H), jnp.float32)],
    )
    return f(mij, dst3, zh)


def _make_scatter_x_body(seg):
    ncnk = NCH_SEG[seg]
    tps = TPS_SEG[seg]

    def body(pf_hbm, dst_hbm, xpart_hbm,
             didx, pbuf0, pbuf1, psem0, psem1, xacc):
        c = lax.axis_index("c")
        s = lax.axis_index("s")
        wid = c * NS + s
        base = wid * tps
        iota16 = lax.iota(jnp.int32, 16)
        pltpu.sync_copy(dst_hbm.at[pl.ds(SEG_OFF[seg] + base, tps)], didx)

        def zloop(i, carry):
            plsc.store_scatter(xacc, [i * 16 + iota16],
                               jnp.zeros((16,), jnp.float32))
            return carry

        lax.fori_loop(0, (N * 4) // 16, zloop, 0)
        pbuf = (pbuf0, pbuf1)
        psem = (psem0, psem1)

        def hoff(j):
            return pl.multiple_of(base + j * CH, 8)

        def issue_load(j, b):
            pltpu.async_copy(pf_hbm.at[pl.ds(hoff(j), CH)], pbuf[b], psem[b])

        def wait_load(j, b):
            pltpu.make_async_copy(pf_hbm.at[pl.ds(hoff(j), CH)],
                                  pbuf[b], psem[b]).wait()

        def compute(j, b):
            for q in range(CH // 16):
                o = pl.multiple_of(j * CH, 8) + q * 16
                d16 = didx[pl.ds(o, 16)]
                r16 = q * 16 + iota16
                for cc in range(3):
                    val = plsc.load_gather(pbuf[b], [r16, iota16 * 0 + cc])
                    plsc.addupdate_scatter(xacc, [d16 * 4 + cc], val)

        issue_load(0, 0)
        npairs = (ncnk - 1) // 2
        rem = ncnk - 2 * npairs

        def dbl(jj, carry):
            j0 = jj * 2
            issue_load(j0 + 1, 1)
            wait_load(j0, 0)
            compute(j0, 0)
            j1 = j0 + 1
            issue_load(j1 + 1, 0)
            wait_load(j1, 1)
            compute(j1, 1)
            return carry

        lax.fori_loop(0, npairs, dbl, 0)
        c0 = 2 * npairs
        if rem == 1:
            wait_load(c0, 0)
            compute(c0, 0)
        else:
            c1 = c0 + 1
            issue_load(c1, 1)
            wait_load(c0, 0)
            compute(c0, 0)
            wait_load(c1, 1)
            compute(c1, 1)
        pltpu.sync_copy(xacc, xpart_hbm.at[wid])

    return body


def _sc_scatter_x(parr, dst, seg):
    tps = TPS_SEG[seg]
    f = pl.kernel(
        _make_scatter_x_body(seg),
        out_type=jax.ShapeDtypeStruct((NW, N * 4), jnp.float32),
        mesh=_sc_mesh(),
        compiler_params=_SC_PARAMS,
        scratch_types=[pltpu.VMEM((tps,), jnp.int32),
                       pltpu.VMEM((CH, PW), jnp.float32),
                       pltpu.VMEM((CH, PW), jnp.float32),
                       pltpu.SemaphoreType.DMA,
                       pltpu.SemaphoreType.DMA,
                       pltpu.VMEM((N * 4,), jnp.float32)],
    )
    return f(parr, dst)


def _xsum_body(x_ref, xp0_ref, xp1_ref, out_ref):
    out_ref[...] = (x_ref[...] + jnp.sum(xp0_ref[...], axis=0)
                    + jnp.sum(xp1_ref[...], axis=0))


def _xsum(x4, xpart0, xpart1):
    XL = 1600
    x3 = x4.reshape(NGRID, 1, XL)
    xp0 = xpart0.reshape(NW, NGRID, 1, XL)
    xp1 = xpart1.reshape(NW, NGRID, 1, XL)
    out = pl.pallas_call(
        _xsum_body,
        grid=(NGRID,),
        in_specs=[pl.BlockSpec((1, 1, XL), lambda i: (i, 0, 0)),
                  pl.BlockSpec((NW, 1, 1, XL), lambda i: (0, i, 0, 0)),
                  pl.BlockSpec((NW, 1, 1, XL), lambda i: (0, i, 0, 0))],
        out_specs=pl.BlockSpec((1, 1, XL), lambda i: (i, 0, 0)),
        out_shape=jax.ShapeDtypeStruct((NGRID, 1, XL), jnp.float32),
    )(x3, xp0, xp1)
    return out.reshape(N, 4)


def _node_body(h_ref, mp0_ref, mp1_ref, U_ref, V_ref, b1_ref,
               W2_ref, b2_ref, hout_ref):
    m_i = mp0_ref[0] + mp0_ref[1] + mp1_ref[0] + mp1_ref[1]
    t = (jnp.dot(h_ref[...], U_ref[...], preferred_element_type=jnp.float32)
         + jnp.dot(m_i, V_ref[...], preferred_element_type=jnp.float32)
         + b1_ref[...])
    hh = t * jax.nn.sigmoid(t)
    hout_ref[...] = (h_ref[...]
                     + jnp.dot(hh, W2_ref[...], preferred_element_type=jnp.float32)
                     + b2_ref[...])


def _node_update(h, mpart0, mpart1, U, V, hb1, hW2, hb2):
    full = lambda shape: pl.BlockSpec(shape, lambda i: tuple(0 for _ in shape))
    return pl.pallas_call(
        _node_body,
        grid=(NGRID,),
        in_specs=[pl.BlockSpec((NB, H), lambda i: (i, 0)),
                  pl.BlockSpec((NC, NB, H), lambda i: (0, i, 0)),
                  pl.BlockSpec((NC, NB, H), lambda i: (0, i, 0)),
                  full((H, H)), full((H, H)), full((1, H)),
                  full((H, H)), full((1, H))],
        out_specs=pl.BlockSpec((NB, H), lambda i: (i, 0)),
        out_shape=jax.ShapeDtypeStruct((N, H), jnp.float32),
    )(h, mpart0, mpart1, U, V, hb1, hW2, hb2)


def kernel(atomic_numbers, pos, edge_index, edge_attr, emb,
           e_w1, e_b1, e_w2, e_b2,
           h_w1, h_b1, h_w2, h_b2,
           x_w1, x_b1, x_w2, x_b2):
    del edge_attr
    ids2d = atomic_numbers.astype(jnp.int32).reshape(N, 1)
    embp = jnp.zeros((H, H), jnp.float32).at[:emb.shape[0]].set(emb)
    src = edge_index[0].astype(jnp.int32)
    dst = edge_index[1].astype(jnp.int32)
    dst3 = [dst[SEG_OFF[s]:SEG_OFF[s] + SEG_E[s]].reshape(NW, NCH_SEG[s], CH)
            for s in range(S)]
    zh = jnp.zeros((N, H), jnp.float32)

    h = _embed(ids2d, embp)
    x4 = jnp.pad(pos, ((0, 0), (0, 1)))

    for l in range(e_w1.shape[0]):
        A = e_w1[l, :H]
        B = e_w1[l, H:2 * H]
        crow = e_w1[l, 2 * H:2 * H + 1]
        b1 = e_b1[l].reshape(1, H)
        W2 = e_w2[l]
        b2 = e_b2[l].reshape(1, H)
        xw1 = x_w1[l]
        xb1 = x_b1[l].reshape(1, H)
        xw2 = x_w2[l]
        xb2 = x_b2[l].reshape(1, 1)
        U = h_w1[l, :H]
        V = h_w1[l, H:]
        hb1 = h_b1[l].reshape(1, H)
        hW2 = h_w2[l]
        hb2 = h_b2[l].reshape(1, H)

        xflat = x4.reshape(-1)
        hi0, hj0, g0 = _sc_gather(h, xflat, src, dst, 0)
        hi1, hj1, g1 = _sc_gather(h, xflat, src, dst, 1)
        mij0, parr0 = _edge_mlp(hi0, hj0, g0, A, B, crow, b1,
                                W2, b2, xw1, xb1, xw2, xb2, 0)
        mij1, parr1 = _edge_mlp(hi1, hj1, g1, A, B, crow, b1,
                                W2, b2, xw1, xb1, xw2, xb2, 1)
        mp0 = _sc_scatter_m(mij0, dst3[0], zh, 0)
        xp0 = _sc_scatter_x(parr0, dst, 0)
        mp1 = _sc_scatter_m(mij1, dst3[1], zh, 1)
        xp1 = _sc_scatter_x(parr1, dst, 1)
        x4 = _xsum(x4, xp0, xp1)
        h = _node_update(h, mp0, mp1, U, V, hb1, hW2, hb2)

    return (h, x4[:, :3])
